# baseline (speedup 1.0000x reference)
import functools

import jax
import jax.numpy as jnp
from jax import lax
from jax.experimental import pallas as pl
from jax.experimental.pallas import tpu as pltpu
from jax.experimental.pallas import tpu_sc as plsc

NUM_SHARDS = 20
ROWS_PER_SHARD = 20000
DIM = 256
HIST = 50
FRAG = 128
NFRAG = 2 * HIST
NPAD = 112
LANES = 16
NC, NS = 2, 16
NW = NC * NS
OB = 16

_i32 = jnp.int32


@functools.partial(jax.jit, static_argnames=("batch",))
def _pooled_lookup(tab, idx, batch):
    rpt = batch // NW
    nblocks = rpt // OB
    mesh = plsc.VectorSubcoreMesh(
        core_axis_name="c", subcore_axis_name="s", num_cores=NC, num_subcores=NS
    )

    @functools.partial(
        pl.kernel,
        out_type=jax.ShapeDtypeStruct((batch, DIM), jnp.float32),
        mesh=mesh,
        scratch_types=[
            pltpu.VMEM((rpt * HIST,), _i32),
            pltpu.VMEM((NPAD,), _i32),
            pltpu.VMEM((NPAD,), _i32),
            pltpu.VMEM((NPAD, FRAG), jnp.float32),
            pltpu.VMEM((NPAD, FRAG), jnp.float32),
            pltpu.VMEM((2, OB, DIM), jnp.float32),
            pltpu.SemaphoreType.DMA,
            pltpu.SemaphoreType.DMA,
            pltpu.SemaphoreType.DMA,
        ],
        compiler_params=pltpu.CompilerParams(needs_layout_passes=False),
    )
    def body(
        tab_hbm,
        idx_hbm,
        out_hbm,
        raw_v,
        ibuf0_v,
        ibuf1_v,
        rows0_v,
        rows1_v,
        obuf_v,
        gsem0,
        gsem1,
        osem,
    ):
        ibufs = (ibuf0_v, ibuf1_v)
        rows = (rows0_v, rows1_v)
        gsems = (gsem0, gsem1)
        wid = lax.axis_index("s") * NC + lax.axis_index("c")
        pltpu.sync_copy(idx_hbm.at[wid], raw_v)

        lane = lax.iota(_i32, LANES)
        twenty = jnp.full((LANES,), NUM_SHARDS, _i32)
        tail_mask = lane < 2
        pad_mask = lane < NPAD - NFRAG

        def conv_row(b, u):
            base = b * HIST
            for off, m in ((0, None), (16, None), (32, None), (48, tail_mask)):
                a = lane + (base + off)
                v = plsc.load_gather(raw_v, [a], mask=m)
                d = lax.rem(v, twenty) * ROWS_PER_SHARD + lax.div(v, twenty)
                f = d * 2
                plsc.store_scatter(ibufs[u], [lane * 2 + 2 * off], f, mask=m)
                plsc.store_scatter(ibufs[u], [lane * 2 + (2 * off + 1)], f + 1, mask=m)
            plsc.store_scatter(ibufs[u], [lane + NFRAG], lane * 0, mask=pad_mask)

        def fire(u):
            pltpu.async_copy(tab_hbm.at[ibufs[u]], rows[u], gsems[u])

        NCHAIN = 4

        def reduce_row(u, pv, ov):
            for g in range(DIM // LANES):
                p, c = (0, g) if g < 8 else (1, g - 8)
                acc = [None] * NCHAIN
                for j in range(HIST):
                    v = rows[u][2 * j + p, pl.ds(c * LANES, LANES)]
                    k = j % NCHAIN
                    acc[k] = v if acc[k] is None else acc[k] + v
                while len(acc) > 1:
                    acc = [
                        acc[i] + acc[i + 1] if i + 1 < len(acc) else acc[i]
                        for i in range(0, len(acc), 2)
                    ]
                plsc.store_scatter(obuf_v, [pv, ov, lane + g * LANES], acc[0])

        conv_row(_i32(0), 0)
        conv_row(_i32(1), 1)
        plsc.subcore_barrier()
        fire(0)
        fire(1)

        def pair(bb, carry):
            for u in (0, 1):
                b = bb * 2 + u
                pltpu.make_async_copy(
                    tab_hbm.at[ibufs[u]], rows[u], gsems[u]
                ).wait()

                @pl.when(bb < rpt // 2 - 1)
                def _():
                    conv_row(b + 2, u)

                omod = lax.rem(b, _i32(OB))
                parity = lax.rem(lax.div(b, _i32(OB)), _i32(2))
                reduce_row(u, lane * 0 + parity, lane * 0 + omod)

                @pl.when(bb < rpt // 2 - 1)
                def _():
                    fire(u)

                if u == 1:

                    @pl.when(omod == OB - 1)
                    def _():
                        pltpu.async_copy(
                            obuf_v.at[parity],
                            out_hbm.at[
                                pl.ds(
                                    pl.multiple_of(wid * rpt + b - (OB - 1), OB),
                                    OB,
                                )
                            ],
                            osem,
                        )

            return carry

        lax.fori_loop(_i32(0), _i32(rpt // 2), pair, _i32(0))

        def drain(i, carry):
            pltpu.make_async_copy(
                obuf_v.at[_i32(0)],
                out_hbm.at[pl.ds(pl.multiple_of(wid * rpt, OB), OB)],
                osem,
            ).wait()
            return carry

        lax.fori_loop(_i32(0), _i32(nblocks), drain, _i32(0))

    return body(tab, idx)


def kernel(inputs, table):
    batch, hist = inputs.shape
    assert hist == HIST and batch % (NW * OB) == 0
    idx = inputs.astype(_i32).reshape(NW, (batch // NW) * HIST)
    tab = table.reshape(NUM_SHARDS * ROWS_PER_SHARD * 2, FRAG)
    return _pooled_lookup(tab, idx, batch)


# speedup vs baseline: 5.9866x; 5.9866x over previous
import functools

import jax
import jax.numpy as jnp
from jax import lax
from jax.experimental import pallas as pl
from jax.experimental.pallas import tpu as pltpu
from jax.experimental.pallas import tpu_sc as plsc

NUM_SHARDS = 20
ROWS_PER_SHARD = 20000
DIM = 256
HIST = 50
FRAG = 128
NFRAG = 2 * HIST
LANES = 16
NC, NS = 2, 16
NW = NC * NS
OB = 16

_i32 = jnp.int32


@functools.partial(jax.jit, static_argnames=("batch",))
def _pooled_lookup(tab, idx, batch):
    rpt = batch // NW
    nblocks = rpt // OB
    mesh = plsc.VectorSubcoreMesh(
        core_axis_name="c", subcore_axis_name="s", num_cores=NC, num_subcores=NS
    )

    @functools.partial(
        pl.kernel,
        out_type=jax.ShapeDtypeStruct((batch, DIM), jnp.float32),
        mesh=mesh,
        scratch_types=[
            pltpu.VMEM((rpt * HIST,), _i32),
            pltpu.VMEM((NFRAG,), _i32),
            pltpu.VMEM((NFRAG,), _i32),
            pltpu.VMEM((NFRAG, FRAG), jnp.float32),
            pltpu.VMEM((NFRAG, FRAG), jnp.float32),
            pltpu.VMEM((2, OB, DIM), jnp.float32),
            pltpu.SemaphoreType.DMA,
            pltpu.SemaphoreType.DMA,
            pltpu.SemaphoreType.DMA,
        ],
        compiler_params=pltpu.CompilerParams(needs_layout_passes=False),
    )
    def body(
        tab_hbm,
        idx_hbm,
        out_hbm,
        raw_v,
        ibuf0_v,
        ibuf1_v,
        rows0_v,
        rows1_v,
        obuf_v,
        gsem0,
        gsem1,
        osem,
    ):
        ibufs = (ibuf0_v, ibuf1_v)
        rows = (rows0_v, rows1_v)
        gsems = (gsem0, gsem1)
        wid = lax.axis_index("s") * NC + lax.axis_index("c")
        pltpu.sync_copy(idx_hbm.at[wid], raw_v)

        lane = lax.iota(_i32, LANES)
        twenty = jnp.full((LANES,), NUM_SHARDS, _i32)
        tail_mask = lane < 2

        def conv_row(b, u):
            base = b * HIST
            for off, m in ((0, None), (16, None), (32, None), (48, tail_mask)):
                a = lane + (base + off)
                v = plsc.load_gather(raw_v, [a], mask=m)
                d = lax.rem(v, twenty) * ROWS_PER_SHARD + lax.div(v, twenty)
                f = d * 2
                plsc.store_scatter(ibufs[u], [lane * 2 + 2 * off], f, mask=m)
                plsc.store_scatter(ibufs[u], [lane * 2 + (2 * off + 1)], f + 1, mask=m)

        def fire(u):
            pltpu.async_copy(tab_hbm.at[ibufs[u]], rows[u], gsems[u])

        NCHAIN = 4

        def reduce_row(u, pv, ov):
            for g in range(DIM // LANES):
                p, c = (0, g) if g < 8 else (1, g - 8)
                acc = [None] * NCHAIN
                for j in range(HIST):
                    v = rows[u][2 * j + p, pl.ds(c * LANES, LANES)]
                    k = j % NCHAIN
                    acc[k] = v if acc[k] is None else acc[k] + v
                while len(acc) > 1:
                    acc = [
                        acc[i] + acc[i + 1] if i + 1 < len(acc) else acc[i]
                        for i in range(0, len(acc), 2)
                    ]
                plsc.store_scatter(obuf_v, [pv, ov, lane + g * LANES], acc[0])

        conv_row(_i32(0), 0)
        conv_row(_i32(1), 1)
        plsc.subcore_barrier()
        fire(0)
        fire(1)

        def pair(bb, carry):
            for u in (0, 1):
                b = bb * 2 + u
                pltpu.make_async_copy(
                    tab_hbm.at[ibufs[u]], rows[u], gsems[u]
                ).wait()

                @pl.when(bb < rpt // 2 - 1)
                def _():
                    conv_row(b + 2, u)

                omod = lax.rem(b, _i32(OB))
                parity = lax.rem(lax.div(b, _i32(OB)), _i32(2))
                reduce_row(u, lane * 0 + parity, lane * 0 + omod)

                @pl.when(bb < rpt // 2 - 1)
                def _():
                    fire(u)

                if u == 1:

                    @pl.when(omod == OB - 1)
                    def _():
                        pltpu.async_copy(
                            obuf_v.at[parity],
                            out_hbm.at[
                                pl.ds(
                                    pl.multiple_of(wid * rpt + b - (OB - 1), OB),
                                    OB,
                                )
                            ],
                            osem,
                        )

            return carry

        lax.fori_loop(_i32(0), _i32(rpt // 2), pair, _i32(0))

        def drain(i, carry):
            pltpu.make_async_copy(
                obuf_v.at[_i32(0)],
                out_hbm.at[pl.ds(pl.multiple_of(wid * rpt, OB), OB)],
                osem,
            ).wait()
            return carry

        lax.fori_loop(_i32(0), _i32(nblocks), drain, _i32(0))

    return body(tab, idx)


def kernel(inputs, table):
    batch, hist = inputs.shape
    assert hist == HIST and batch % (NW * OB) == 0
    idx = inputs.astype(_i32).reshape(NW, (batch // NW) * HIST)
    tab = table.reshape(NUM_SHARDS * ROWS_PER_SHARD * 2, FRAG)
    return _pooled_lookup(tab, idx, batch)
